# Initial kernel scaffold; baseline (speedup 1.0000x reference)
import jax
import jax.numpy as jnp
from jax.experimental import pallas as pl


def kernel(data, neigh, depth, weight, gamma, beta):
    raise NotImplementedError("write your pallas kernel here")


# speedup vs baseline: 3.8799x; 3.8799x over previous
import functools

import jax
import jax.numpy as jnp
from jax import lax
from jax.experimental import pallas as pl
from jax.experimental.pallas import tpu as pltpu
from jax.experimental.pallas import tpu_sc as plsc

N = 10000
C_IN = 256
C_OUT = 256
K = 27
BN_EPS = 1e-5

E = N * K
NUM_TILES = 16
CHUNK = 128
E_TILE = 16896
E_PAD = NUM_TILES * E_TILE
NCHUNK = E_TILE // CHUNK
HALF = N // 2
HALF_PAD = 5008
STRIPE = 320
LAST_STRIPE = HALF_PAD - (NUM_TILES - 1) * STRIPE


def _matmul_tc(data, weight):
    def body(d_ref, w_ref, o_ref):
        o_ref[...] = jnp.dot(d_ref[...], w_ref[0],
                             preferred_element_type=jnp.float32)

    return pl.pallas_call(
        body,
        grid=(K,),
        in_specs=[
            pl.BlockSpec((N, C_IN), lambda k: (0, 0)),
            pl.BlockSpec((1, C_IN, C_OUT), lambda k: (k, 0, 0)),
        ],
        out_specs=pl.BlockSpec((N, C_OUT), lambda k: (k, 0)),
        out_shape=jax.ShapeDtypeStruct((E_PAD, C_OUT), jnp.float32),
    )(data, weight)


def _scatter_sc(contrib, idx, zeros):
    mesh = plsc.VectorSubcoreMesh(core_axis_name="c", subcore_axis_name="s")

    @functools.partial(
        pl.kernel,
        out_type=jax.ShapeDtypeStruct((2, HALF_PAD, C_OUT), jnp.float32),
        mesh=mesh,
        compiler_params=pltpu.CompilerParams(use_tc_tiling_on_sc=False),
        scratch_types=[
            pltpu.VMEM((CHUNK,), jnp.int32),
            pltpu.VMEM((CHUNK, C_OUT), jnp.float32),
            pltpu.VMEM_SHARED((HALF_PAD, C_OUT), jnp.float32),
        ],
    )
    def body(contrib_hbm, idx_hbm, zeros_hbm, out_hbm, cidx_v, buf_v, acc_sh):
        c = lax.axis_index("c")
        s = lax.axis_index("s")
        lo = c * HALF

        @pl.when(s < NUM_TILES - 1)
        def _():
            pltpu.sync_copy(zeros_hbm.at[pl.ds(0, STRIPE)],
                            acc_sh.at[pl.ds(s * STRIPE, STRIPE)])

        @pl.when(s == NUM_TILES - 1)
        def _():
            pltpu.sync_copy(zeros_hbm.at[pl.ds(0, LAST_STRIPE)],
                            acc_sh.at[pl.ds(s * STRIPE, LAST_STRIPE)])

        plsc.subcore_barrier()

        base = s * E_TILE

        def scatter_chunk(it, _):
            pltpu.sync_copy(idx_hbm.at[s, it], cidx_v)
            pltpu.sync_copy(contrib_hbm.at[pl.ds(base + it * CHUNK, CHUNK)],
                            buf_v)
            for j in range(CHUNK // 16):
                v = cidx_v[pl.ds(j * 16, 16)]
                ok = (v >= lo) & (v < lo + HALF)
                cidx_v[pl.ds(j * 16, 16)] = jnp.where(ok, v - lo, HALF)
            pltpu.sync_copy(buf_v, acc_sh.at[cidx_v], add=True)
            return 0

        lax.fori_loop(0, NCHUNK, scatter_chunk, 0)
        plsc.subcore_barrier()

        @pl.when(s < NUM_TILES - 1)
        def _():
            pltpu.sync_copy(acc_sh.at[pl.ds(s * STRIPE, STRIPE)],
                            out_hbm.at[c, pl.ds(s * STRIPE, STRIPE)])

        @pl.when(s == NUM_TILES - 1)
        def _():
            pltpu.sync_copy(acc_sh.at[pl.ds(s * STRIPE, LAST_STRIPE)],
                            out_hbm.at[c, pl.ds(s * STRIPE, LAST_STRIPE)])

    return body(contrib, idx, zeros)


def _bn_gelu_tc(x, gamma, beta):
    def body(x_ref, g_ref, b_ref, o_ref):
        v = x_ref[...]
        mean = jnp.mean(v, axis=0, keepdims=True)
        var = jnp.mean((v - mean) ** 2, axis=0, keepdims=True)
        xhat = (v - mean) * lax.rsqrt(var + BN_EPS)
        y = xhat * g_ref[...] + b_ref[...]
        o_ref[...] = 0.5 * y * (1.0 + lax.erf(y * 0.7071067811865476))

    return pl.pallas_call(
        body,
        out_shape=jax.ShapeDtypeStruct((N, C_OUT), jnp.float32),
    )(x, gamma.reshape(1, C_OUT), beta.reshape(1, C_OUT))


def kernel(data, neigh, depth, weight, gamma, beta):
    del depth
    contrib = _matmul_tc(data, weight)

    idx_flat = neigh.T.reshape(-1)
    idx = jnp.concatenate(
        [idx_flat, jnp.full((E_PAD - E,), -1, jnp.int32)]
    ).reshape(NUM_TILES, NCHUNK, CHUNK)

    zeros = jnp.zeros((STRIPE, C_OUT), jnp.float32)
    halves = _scatter_sc(contrib, idx, zeros)
    out = jnp.concatenate([halves[0, :HALF], halves[1, :HALF]], axis=0)
    return _bn_gelu_tc(out, gamma, beta)


# speedup vs baseline: 6.1701x; 1.5903x over previous
import functools

import jax
import jax.numpy as jnp
from jax import lax
from jax.experimental import pallas as pl
from jax.experimental.pallas import tpu as pltpu
from jax.experimental.pallas import tpu_sc as plsc

N = 10000
C_IN = 256
C_OUT = 256
K = 27
BN_EPS = 1e-5

E = N * K
NUM_TILES = 16
CHUNK = 128
NCHUNK = 66
E_TILE = NCHUNK * CHUNK
E_SC = NUM_TILES * E_TILE
E_PAD = 2 * E_SC
DUMP = N
ACC_ROWS = 10112
STRIPE = ACC_ROWS // NUM_TILES


def _matmul_tc(data, weight):
    def body(d_ref, w_ref, o_ref):
        o_ref[...] = jnp.dot(
            d_ref[...], w_ref[0], preferred_element_type=jnp.float32
        ).astype(jnp.bfloat16)

    return pl.pallas_call(
        body,
        grid=(K,),
        in_specs=[
            pl.BlockSpec((N, C_IN), lambda k: (0, 0)),
            pl.BlockSpec((1, C_IN, C_OUT), lambda k: (k, 0, 0)),
        ],
        out_specs=pl.BlockSpec((N, C_OUT), lambda k: (k, 0)),
        out_shape=jax.ShapeDtypeStruct((E_PAD, C_OUT), jnp.bfloat16),
    )(data.astype(jnp.bfloat16), weight.astype(jnp.bfloat16))


def _scatter_sc(contrib, idx, zeros):
    mesh = plsc.VectorSubcoreMesh(core_axis_name="c", subcore_axis_name="s")

    @functools.partial(
        pl.kernel,
        out_type=jax.ShapeDtypeStruct((2, ACC_ROWS, C_OUT), jnp.bfloat16),
        mesh=mesh,
        compiler_params=pltpu.CompilerParams(use_tc_tiling_on_sc=False),
        scratch_types=[
            pltpu.VMEM((CHUNK,), jnp.int32),
            pltpu.VMEM((CHUNK, C_OUT), jnp.bfloat16),
            pltpu.VMEM_SHARED((ACC_ROWS, C_OUT), jnp.bfloat16),
        ],
    )
    def body(contrib_hbm, idx_hbm, zeros_hbm, out_hbm, cidx_v, buf_v, acc_sh):
        c = lax.axis_index("c")
        s = lax.axis_index("s")

        pltpu.sync_copy(zeros_hbm, acc_sh.at[pl.ds(s * STRIPE, STRIPE)])
        plsc.subcore_barrier()

        base = c * E_SC + s * E_TILE

        def scatter_chunk(it, _):
            pltpu.sync_copy(idx_hbm.at[c, s, it], cidx_v)
            pltpu.sync_copy(contrib_hbm.at[pl.ds(base + it * CHUNK, CHUNK)],
                            buf_v)
            pltpu.sync_copy(buf_v, acc_sh.at[cidx_v], add=True)
            return 0

        lax.fori_loop(0, NCHUNK, scatter_chunk, 0)
        plsc.subcore_barrier()

        pltpu.sync_copy(acc_sh.at[pl.ds(s * STRIPE, STRIPE)],
                        out_hbm.at[c, pl.ds(s * STRIPE, STRIPE)])

    return body(contrib, idx, zeros)


def _bn_gelu_tc(a0, a1, gamma, beta):
    def body(a0_ref, a1_ref, g_ref, b_ref, o_ref):
        v = a0_ref[...].astype(jnp.float32) + a1_ref[...].astype(jnp.float32)
        mean = jnp.mean(v, axis=0, keepdims=True)
        var = jnp.mean((v - mean) ** 2, axis=0, keepdims=True)
        xhat = (v - mean) * lax.rsqrt(var + BN_EPS)
        y = xhat * g_ref[...] + b_ref[...]
        o_ref[...] = 0.5 * y * (1.0 + lax.erf(y * 0.7071067811865476))

    return pl.pallas_call(
        body,
        out_shape=jax.ShapeDtypeStruct((N, C_OUT), jnp.float32),
    )(a0, a1, gamma.reshape(1, C_OUT), beta.reshape(1, C_OUT))


def kernel(data, neigh, depth, weight, gamma, beta):
    del depth
    contrib = _matmul_tc(data, weight)

    idx_flat = neigh.T.reshape(-1)
    idx = jnp.concatenate(
        [idx_flat, jnp.full((E_PAD - E,), DUMP, jnp.int32)]
    ).reshape(2, NUM_TILES, NCHUNK, CHUNK)

    zeros = jnp.zeros((STRIPE, C_OUT), jnp.bfloat16)
    partials = _scatter_sc(contrib, idx, zeros)
    return _bn_gelu_tc(partials[0, :N], partials[1, :N], gamma, beta)
